# Initial kernel scaffold; baseline (speedup 1.0000x reference)
import jax
import jax.numpy as jnp
from jax.experimental import pallas as pl


def kernel(seq_a, seq_b):
    raise NotImplementedError("write your pallas kernel here")


# speedup vs baseline: 10.4280x; 10.4280x over previous
import functools

import jax
import jax.numpy as jnp
from jax import lax
from jax.experimental import pallas as pl
from jax.experimental.pallas import tpu as pltpu
from jax.experimental.pallas import tpu_sc as plsc

B, N, M, F = 64, 128, 128, 16
INF = 1000000.0
LN2 = 0.6931471805599453
NCHUNK = 9
W = NCHUNK * 16


def _dist_body(a_ref, b_ref, o_ref):
    a = a_ref[...]
    b = b_ref[...]
    ab = lax.dot_general(a, b, (((2,), (2,)), ((0,), (0,))),
                         preferred_element_type=jnp.float32)
    na = jnp.sum(a * a, axis=-1)
    nb = jnp.sum(b * b, axis=-1)
    o_ref[...] = na[:, :, None] + nb[:, None, :] - 2.0 * ab


def _dist(a, b):
    return pl.pallas_call(
        _dist_body,
        out_shape=jax.ShapeDtypeStruct((B, N, M), jnp.float32),
    )(a, b)


def _ln(z):
    zi = plsc.bitcast(z, jnp.int32)
    e = lax.shift_right_logical(zi, 23) - 127
    mant = plsc.bitcast((zi & 0x007FFFFF) | 0x3F800000, jnp.float32)
    big = mant > 1.4142135
    mant = jnp.where(big, 0.5 * mant, mant)
    e = e + jnp.where(big, 1, 0)
    s = (mant - 1.0) / (mant + 1.0)
    u = s * s
    poly = ((u * (1.0 / 7.0) + (1.0 / 5.0)) * u + (1.0 / 3.0)) * u + 1.0
    return e.astype(jnp.float32) * LN2 + 2.0 * s * poly


@functools.partial(
    pl.kernel,
    out_type=jax.ShapeDtypeStruct((B, 16), jnp.float32),
    mesh=plsc.VectorSubcoreMesh(core_axis_name="c", subcore_axis_name="s"),
    compiler_params=pltpu.CompilerParams(needs_layout_passes=False),
    scratch_types=[
        pltpu.VMEM((N * M,), jnp.float32),
        pltpu.VMEM((W,), jnp.float32),
        pltpu.VMEM((W,), jnp.float32),
        pltpu.VMEM((W,), jnp.float32),
        pltpu.VMEM((16,), jnp.float32),
    ],
)
def _sc_dp(d_hbm, out_hbm, d_v, p0, p1, p2, o_v):
    nc = plsc.get_sparse_core_info().num_cores
    wid = lax.axis_index("s") * nc + lax.axis_index("c")
    iota = lax.iota(jnp.int32, 16)
    inf_v = jnp.full((16,), INF, jnp.float32)

    def diag_step(t, cur, prev1, prev2, d_v):
        lo = jnp.maximum(1, t - M)
        hi = jnp.minimum(N, t - 1)
        for c in range(NCHUNK):
            ivec = c * 16 + iota
            r_up = prev1[pl.ds(c * 16, 16)]
            ish = jnp.maximum(ivec - 1, 0)
            r_left = plsc.load_gather(prev1, [ish])
            r_dd = plsc.load_gather(prev2, [ish])
            didx = jnp.clip(127 * ivec + (t - 129), 0, N * M - 1)
            d = plsc.load_gather(d_v, [didx])
            m3 = jnp.minimum(jnp.minimum(r_left, r_up), r_dd)
            z = (jnp.exp(m3 - r_left) + jnp.exp(m3 - r_up) +
                 jnp.exp(m3 - r_dd))
            val = d + (m3 - _ln(z))
            valid = (ivec >= lo) & (ivec <= hi)
            cur[pl.ds(c * 16, 16)] = jnp.where(valid, val, INF)

    for p in range(2):
        pair = wid * 2 + p
        pltpu.sync_copy(d_hbm.at[pair], d_v)
        for c in range(NCHUNK):
            sl = pl.ds(c * 16, 16)
            p0[sl] = jnp.where(iota == 0, 0.0, INF) if c == 0 else inf_v
            p1[sl] = inf_v
            p2[sl] = inf_v

        def body(k, carry):
            t = 3 * k + 2
            diag_step(t, p2, p1, p0, d_v)
            diag_step(t + 1, p0, p2, p1, d_v)
            diag_step(t + 2, p1, p0, p2, d_v)
            return carry

        lax.fori_loop(0, (N + M - 1) // 3, body, 0)
        o_v[...] = p1[pl.ds(128, 16)]
        pltpu.sync_copy(o_v, out_hbm.at[pair])


def kernel(seq_a, seq_b):
    d = _dist(seq_a, seq_b).reshape(B, N * M)
    out = _sc_dp(d)
    return out[:, 0:1]
